# baseline (speedup 1.0000x reference)
import dataclasses
import functools

import jax
import jax.numpy as jnp
from jax import lax
from jax.experimental import pallas as pl
from jax.experimental.pallas import tpu as pltpu
from jax.experimental.pallas import tpu_sc as plsc

_B = 8
_NCLS = 40
_ALPHA = 0.2
_NW = 32
_L = 16


def _lrelu(x):
    return jnp.where(x >= 0, x, _ALPHA * x)


def _fps_body(npoint, x_ref, y_ref, z_ref, nx_ref, ny_ref, nz_ref):
    X, Y, Z = x_ref[...], y_ref[...], z_ref[...]
    B, N = X.shape
    lane = lax.broadcasted_iota(jnp.int32, (B, N), 1)
    neg = jnp.float32(-3.0e38)
    big = jnp.int32(N)

    def step(t, carry):
        dist, far = carry
        eq = lane == far
        cx = jnp.max(jnp.where(eq, X, neg), axis=1, keepdims=True)
        cy = jnp.max(jnp.where(eq, Y, neg), axis=1, keepdims=True)
        cz = jnp.max(jnp.where(eq, Z, neg), axis=1, keepdims=True)
        nx_ref[pl.ds(t, 1), :] = cx.reshape(1, B)
        ny_ref[pl.ds(t, 1), :] = cy.reshape(1, B)
        nz_ref[pl.ds(t, 1), :] = cz.reshape(1, B)
        dx = X - cx
        dy = Y - cy
        dz = Z - cz
        d = (dx * dx + dy * dy) + dz * dz
        dist = jnp.minimum(dist, d)
        m = jnp.max(dist, axis=1, keepdims=True)
        new_far = jnp.min(jnp.where(dist == m, lane, big), axis=1, keepdims=True)
        return dist, new_far

    init = (jnp.full((B, N), 1e10, dtype=jnp.float32),
            jnp.zeros((B, 1), dtype=jnp.int32))
    lax.fori_loop(0, npoint, step, init, unroll=2)


def _fps(xyz, npoint):
    B, N, _ = xyz.shape
    xt = jnp.transpose(xyz, (2, 0, 1))
    outs = pl.pallas_call(
        functools.partial(_fps_body, npoint),
        out_shape=[jax.ShapeDtypeStruct((npoint, B), jnp.float32)] * 3,
    )(xt[0], xt[1], xt[2])
    return jnp.stack([o.T for o in outs], axis=-1)


def _mm_body(x_ref, w_ref, b_ref, o_ref):
    o_ref[...] = x_ref[...] @ w_ref[...] + b_ref[...]


def _mm(x, w, b, tile=4096):
    R, Cin = x.shape
    C = w.shape[1]
    tile = min(tile, R)
    return pl.pallas_call(
        _mm_body,
        grid=(R // tile,),
        in_specs=[pl.BlockSpec((tile, Cin), lambda i: (i, 0)),
                  pl.BlockSpec((Cin, C), lambda i: (0, 0)),
                  pl.BlockSpec((1, C), lambda i: (0, 0))],
        out_specs=pl.BlockSpec((tile, C), lambda i: (i, 0)),
        out_shape=jax.ShapeDtypeStruct((R, C), jnp.float32),
    )(x, w, b.reshape(1, C))


def _sc_select_gather(N, S, Ks, r2s, xf, yf, zf, qxf, qyf, qzf, tables):
    R = _B * S
    rows_per = R // _NW
    nch = N // _L
    Cs = tuple(int(t.shape[1]) for t in tables)
    mesh = plsc.VectorSubcoreMesh(core_axis_name="c", subcore_axis_name="s")
    out_type = [jax.ShapeDtypeStruct((R * K, C), jnp.float32)
                for K, C in zip(Ks, Cs)]
    scratch = ([pltpu.VMEM((N,), jnp.float32)] * 3
               + [pltpu.VMEM((rows_per,), jnp.float32)] * 3
               + [pltpu.VMEM((K + _L,), jnp.int32) for K in Ks]
               + [pltpu.VMEM((K,), jnp.int32) for K in Ks]
               + [pltpu.VMEM((K, C), jnp.float32) for K, C in zip(Ks, Cs)]
               + [pltpu.SemaphoreType.DMA] * 3)

    cp = pltpu.CompilerParams(needs_layout_passes=False,
                              use_tc_tiling_on_sc=False)

    @functools.partial(pl.kernel, mesh=mesh, out_type=out_type,
                       scratch_types=scratch, compiler_params=cp)
    def body(x_hbm, y_hbm, z_hbm, qx_hbm, qy_hbm, qz_hbm,
             t0_hbm, t1_hbm, t2_hbm, o0_hbm, o1_hbm, o2_hbm,
             Xv, Yv, Zv, Qx, Qy, Qz, g0, g1, g2, i0, i1, i2,
             r0, r1, r2, s0, s1, s2):
        wid = lax.axis_index("s") * 2 + lax.axis_index("c")
        base = wid * rows_per
        b = base // S
        pltpu.sync_copy(x_hbm.at[pl.ds(b * N, N)], Xv)
        pltpu.sync_copy(y_hbm.at[pl.ds(b * N, N)], Yv)
        pltpu.sync_copy(z_hbm.at[pl.ds(b * N, N)], Zv)
        pltpu.sync_copy(qx_hbm.at[pl.ds(base, rows_per)], Qx)
        pltpu.sync_copy(qy_hbm.at[pl.ds(base, rows_per)], Qy)
        pltpu.sync_copy(qz_hbm.at[pl.ds(base, rows_per)], Qz)
        lane = lax.iota(jnp.int32, _L)
        gbufs = (g0, g1, g2)
        ibufs = (i0, i1, i2)
        rbufs = (r0, r1, r2)
        sems = (s0, s1, s2)
        tabs = (t0_hbm, t1_hbm, t2_hbm)
        outs = (o0_hbm, o1_hbm, o2_hbm)

        @pl.loop(0, rows_per)
        def _row(r):
            cb = (r // _L) * _L
            sel = lane == (r - cb)
            qx = jnp.full((_L,), jnp.sum(jnp.where(sel, Qx[pl.ds(cb, _L)], 0.0)))
            qy = jnp.full((_L,), jnp.sum(jnp.where(sel, Qy[pl.ds(cb, _L)], 0.0)))
            qz = jnp.full((_L,), jnp.sum(jnp.where(sel, Qz[pl.ds(cb, _L)], 0.0)))

            def chunk(i, ws):
                dx = Xv[pl.ds(i * _L, _L)] - qx
                dy = Yv[pl.ds(i * _L, _L)] - qy
                dz = Zv[pl.ds(i * _L, _L)] - qz
                d = (dx * dx + dy * dy) + dz * dz
                gvec = lane + (i * _L + b * N)
                new_ws = []
                for j in range(3):
                    m = d <= r2s[j]
                    mi = m.astype(jnp.int32)
                    cnt = jnp.sum(mi)
                    w = ws[j]
                    pl.when((cnt > 0) & (w < Ks[j]))(
                        lambda j=j, w=w, m=m, mi=mi: plsc.store_scatter(
                            gbufs[j], [plsc.cumsum(mi) + (w - 1)], gvec,
                            mask=m))
                    new_ws.append(jnp.where(w < Ks[j], w + cnt, w))
                return tuple(new_ws)

            ws = lax.fori_loop(0, nch, chunk, (jnp.int32(0),) * 3)
            rid = base + r
            handles = []
            for j in range(3):
                K = Ks[j]
                chunk0 = gbufs[j][pl.ds(0, _L)]
                first = jnp.full((_L,), jnp.sum(
                    jnp.where(lane == 0, chunk0, jnp.int32(0))))
                for c in range(K // _L):
                    cur = gbufs[j][pl.ds(c * _L, _L)]
                    keep = (lane + (c * _L)) < ws[j]
                    ibufs[j][pl.ds(c * _L, _L)] = jnp.where(keep, cur, first)
                handles.append(pltpu.async_copy(
                    tabs[j].at[ibufs[j]], rbufs[j], sems[j]))
            for j in range(3):
                handles[j].wait()
                pltpu.sync_copy(rbufs[j], outs[j].at[pl.ds(rid * Ks[j], Ks[j])])

    return body(xf, yf, zf, qxf, qyf, qzf, *tables)


def _mlp_body(K, rows_ref, q_ref, w2, b2, w3, b3, o_ref):
    T = q_ref.shape[0]
    C1 = rows_ref.shape[1]
    g = rows_ref[...].reshape(T, K, C1)
    h1 = _lrelu(g - q_ref[...][:, None, :])
    h2 = _lrelu(h1.reshape(T * K, C1) @ w2[...] + b2[...])
    h3 = _lrelu(h2 @ w3[...] + b3[...])
    C3 = h3.shape[1]
    o_ref[...] = jnp.max(h3.reshape(T, K, C3), axis=1)


def _branch_mlp(rows, q, W2, b2, W3, b3, K):
    R = q.shape[0]
    C1 = rows.shape[1]
    C2 = W2.shape[1]
    C3 = W3.shape[1]
    T = 2048 // K
    return pl.pallas_call(
        functools.partial(_mlp_body, K),
        grid=(R // T,),
        in_specs=[pl.BlockSpec((T * K, C1), lambda i: (i, 0)),
                  pl.BlockSpec((T, C1), lambda i: (i, 0)),
                  pl.BlockSpec((C1, C2), lambda i: (0, 0)),
                  pl.BlockSpec((1, C2), lambda i: (0, 0)),
                  pl.BlockSpec((C2, C3), lambda i: (0, 0)),
                  pl.BlockSpec((1, C3), lambda i: (0, 0))],
        out_specs=pl.BlockSpec((T, C3), lambda i: (i, 0)),
        out_shape=jax.ShapeDtypeStruct((R, C3), jnp.float32),
    )(rows, q, W2, b2.reshape(1, C2), W3, b3.reshape(1, C3))


def _tail_body(xyz_ref, pts_ref, w31, b31, w32, b32, w33, b33,
               wd1, bd1, wd2, bd2, wd3, bd3, out_ref):
    B, S, _ = xyz_ref.shape
    feat = jnp.concatenate([xyz_ref[...], pts_ref[...]], axis=-1)
    feat = feat.reshape(B * S, feat.shape[-1])
    h = _lrelu(feat @ w31[...] + b31[...])
    h = _lrelu(h @ w32[...] + b32[...])
    h = _lrelu(h @ w33[...] + b33[...])
    net = jnp.max(h.reshape(B, S, h.shape[-1]), axis=1)
    net = _lrelu(net @ wd1[...] + bd1[...])
    net = _lrelu(net @ wd2[...] + bd2[...])
    logits = net @ wd3[...] + bd3[...]
    m = jnp.max(logits, axis=-1, keepdims=True)
    e = jnp.exp(logits - m)
    out_ref[...] = e / jnp.sum(e, axis=-1, keepdims=True)


def _tail(xyz2, pts2, params):
    (w31, b31), (w32, b32), (w33, b33) = params['l3']
    (wd1, bd1), = params['d1']
    (wd2, bd2), = params['d2']
    wd3, bd3 = params['d3']
    weights = (w31, b31[None, :], w32, b32[None, :], w33, b33[None, :],
               wd1, bd1[None, :], wd2, bd2[None, :], wd3, bd3[None, :])
    return pl.pallas_call(
        _tail_body,
        out_shape=jax.ShapeDtypeStruct((_B, _NCLS), jnp.float32),
    )(xyz2, pts2, *weights)


def _pad4(w):
    return jnp.concatenate([w, jnp.zeros((1, w.shape[1]), w.dtype)], axis=0)


def _sa_layer(xyz, feats, npoint, radii, Ks, branch_params):
    B, N, _ = xyz.shape
    new_xyz = _fps(xyz, npoint)
    S = npoint
    xflat = xyz.reshape(B * N, 3)
    if feats is None:
        pin = jnp.concatenate([xflat, jnp.zeros((B * N, 1), jnp.float32)], -1)
    else:
        pin = jnp.concatenate([xflat, feats], axis=-1)
    qflat = new_xyz.reshape(B * S, 3)
    qpad = jnp.concatenate([qflat, jnp.zeros((B * S, 1), jnp.float32)], -1)

    tables = []
    Qs = []
    for (W1, b1), _, _ in branch_params:
        W1in = _pad4(W1) if feats is None else W1
        tables.append(_mm(pin, W1in, b1))
        Qs.append(_mm(qpad, _pad4(W1[:3]), jnp.zeros((W1.shape[1],), jnp.float32)))

    xt = jnp.transpose(xyz, (2, 0, 1)).reshape(3, B * N)
    qt = jnp.transpose(new_xyz, (2, 0, 1)).reshape(3, B * S)
    r2s = tuple(float(r) * float(r) for r in radii)
    rows = _sc_select_gather(N, S, tuple(Ks), r2s,
                             xt[0], xt[1], xt[2], qt[0], qt[1], qt[2], tables)
    outs = []
    for j, ((W1, b1), (W2, b2), (W3, b3)) in enumerate(branch_params):
        outs.append(_branch_mlp(rows[j], Qs[j], W2, b2, W3, b3, Ks[j]))
    return new_xyz, jnp.concatenate(outs, axis=-1)


def kernel(input, params):
    xyz1, pts1 = _sa_layer(input, None, 1024, [0.1, 0.2, 0.4], [16, 32, 128],
                           params['l1'])
    xyz2, pts2 = _sa_layer(xyz1, pts1, 512, [0.2, 0.4, 0.8], [32, 64, 128],
                           params['l2'])
    S2 = xyz2.shape[1]
    return _tail(xyz2, pts2.reshape(_B, S2, pts2.shape[-1]), params)


# speedup vs baseline: 1.2433x; 1.2433x over previous
import dataclasses
import functools

import jax
import jax.numpy as jnp
from jax import lax
from jax.experimental import pallas as pl
from jax.experimental.pallas import tpu as pltpu
from jax.experimental.pallas import tpu_sc as plsc

_B = 8
_NCLS = 40
_ALPHA = 0.2
_NW = 32
_L = 16


def _lrelu(x):
    return jnp.where(x >= 0, x, _ALPHA * x)


def _fps_body(npoint, x_ref, y_ref, z_ref, nx_ref, ny_ref, nz_ref):
    X, Y, Z = x_ref[...], y_ref[...], z_ref[...]
    B, N = X.shape
    lane = lax.broadcasted_iota(jnp.int32, (B, N), 1)
    neg = jnp.float32(-3.0e38)
    big = jnp.int32(N)

    def step(t, carry):
        dist, far = carry
        eq = lane == far
        cx = jnp.max(jnp.where(eq, X, neg), axis=1, keepdims=True)
        cy = jnp.max(jnp.where(eq, Y, neg), axis=1, keepdims=True)
        cz = jnp.max(jnp.where(eq, Z, neg), axis=1, keepdims=True)
        nx_ref[pl.ds(t, 1), :] = cx.reshape(1, B)
        ny_ref[pl.ds(t, 1), :] = cy.reshape(1, B)
        nz_ref[pl.ds(t, 1), :] = cz.reshape(1, B)
        dx = X - cx
        dy = Y - cy
        dz = Z - cz
        d = (dx * dx + dy * dy) + dz * dz
        dist = jnp.minimum(dist, d)
        m = jnp.max(dist, axis=1, keepdims=True)
        new_far = jnp.min(jnp.where(dist == m, lane, big), axis=1, keepdims=True)
        return dist, new_far

    init = (jnp.full((B, N), 1e10, dtype=jnp.float32),
            jnp.zeros((B, 1), dtype=jnp.int32))
    lax.fori_loop(0, npoint, step, init, unroll=2)


def _fps(xyz, npoint):
    B, N, _ = xyz.shape
    xt = jnp.transpose(xyz, (2, 0, 1))
    outs = pl.pallas_call(
        functools.partial(_fps_body, npoint),
        out_shape=[jax.ShapeDtypeStruct((npoint, B), jnp.float32)] * 3,
    )(xt[0], xt[1], xt[2])
    return jnp.stack([o.T for o in outs], axis=-1)


def _mm_body(x_ref, w_ref, b_ref, o_ref):
    o_ref[...] = x_ref[...] @ w_ref[...] + b_ref[...]


def _mm(x, w, b, tile=4096):
    R, Cin = x.shape
    C = w.shape[1]
    tile = min(tile, R)
    return pl.pallas_call(
        _mm_body,
        grid=(R // tile,),
        in_specs=[pl.BlockSpec((tile, Cin), lambda i: (i, 0)),
                  pl.BlockSpec((Cin, C), lambda i: (0, 0)),
                  pl.BlockSpec((1, C), lambda i: (0, 0))],
        out_specs=pl.BlockSpec((tile, C), lambda i: (i, 0)),
        out_shape=jax.ShapeDtypeStruct((R, C), jnp.float32),
    )(x, w, b.reshape(1, C))


def _sc_select_gather(N, S, Ks, r2s, xf, yf, zf, qxf, qyf, qzf, tables):
    R = _B * S
    rows_per = R // _NW
    nch = N // _L
    Cs = tuple(int(t.shape[1]) for t in tables)
    mesh = plsc.VectorSubcoreMesh(core_axis_name="c", subcore_axis_name="s")
    out_type = [jax.ShapeDtypeStruct((R * K, C), jnp.float32)
                for K, C in zip(Ks, Cs)]
    scratch = ([pltpu.VMEM((N,), jnp.float32)] * 3
               + [pltpu.VMEM((rows_per,), jnp.float32)] * 3
               + [pltpu.VMEM((K + _L,), jnp.int32) for K in Ks]
               + [pltpu.VMEM((2, K), jnp.int32) for K in Ks]
               + [pltpu.VMEM((2, K, C), jnp.float32) for K, C in zip(Ks, Cs)]
               + [pltpu.SemaphoreType.DMA] * 2)

    cp = pltpu.CompilerParams(needs_layout_passes=False,
                              use_tc_tiling_on_sc=False)

    @functools.partial(pl.kernel, mesh=mesh, out_type=out_type,
                       scratch_types=scratch, compiler_params=cp)
    def body(x_hbm, y_hbm, z_hbm, qx_hbm, qy_hbm, qz_hbm,
             t0_hbm, t1_hbm, t2_hbm, o0_hbm, o1_hbm, o2_hbm,
             Xv, Yv, Zv, Qx, Qy, Qz, g0, g1, g2, i0, i1, i2,
             r0, r1, r2, s0, s1):
        wid = lax.axis_index("s") * 2 + lax.axis_index("c")
        base = wid * rows_per
        b = base // S
        pltpu.sync_copy(x_hbm.at[pl.ds(b * N, N)], Xv)
        pltpu.sync_copy(y_hbm.at[pl.ds(b * N, N)], Yv)
        pltpu.sync_copy(z_hbm.at[pl.ds(b * N, N)], Zv)
        pltpu.sync_copy(qx_hbm.at[pl.ds(base, rows_per)], Qx)
        pltpu.sync_copy(qy_hbm.at[pl.ds(base, rows_per)], Qy)
        pltpu.sync_copy(qz_hbm.at[pl.ds(base, rows_per)], Qz)
        lane = lax.iota(jnp.int32, _L)
        gbufs = (g0, g1, g2)
        ibufs = (i0, i1, i2)
        rbufs = (r0, r1, r2)
        sems = (s0, s1)
        tabs = (t0_hbm, t1_hbm, t2_hbm)
        outs = (o0_hbm, o1_hbm, o2_hbm)

        def _gathers(slot):
            return [pltpu.make_async_copy(
                tabs[j].at[ibufs[j].at[slot]], rbufs[j].at[slot], sems[slot])
                for j in range(3)]

        def _half(r, slot):
            cb = (r // _L) * _L
            sel = lane == (r - cb)
            qx = jnp.full((_L,), jnp.sum(jnp.where(sel, Qx[pl.ds(cb, _L)], 0.0)))
            qy = jnp.full((_L,), jnp.sum(jnp.where(sel, Qy[pl.ds(cb, _L)], 0.0)))
            qz = jnp.full((_L,), jnp.sum(jnp.where(sel, Qz[pl.ds(cb, _L)], 0.0)))

            def chunk(i, ws):
                dx = Xv[pl.ds(i * _L, _L)] - qx
                dy = Yv[pl.ds(i * _L, _L)] - qy
                dz = Zv[pl.ds(i * _L, _L)] - qz
                d = (dx * dx + dy * dy) + dz * dz
                gvec = lane + (i * _L + b * N)
                new_ws = []
                for j in range(3):
                    m = d <= r2s[j]
                    mi = m.astype(jnp.int32)
                    incl = plsc.cumsum(mi)
                    cnt = jnp.max(incl)
                    w = ws[j]
                    pos = incl + (w - 1)
                    pl.when(w < Ks[j])(
                        lambda j=j, pos=pos, m=m: plsc.store_scatter(
                            gbufs[j], [pos], gvec, mask=m))
                    new_ws.append(jnp.where(w < Ks[j], w + cnt, w))
                return tuple(new_ws)

            ws = lax.fori_loop(0, nch, chunk, (jnp.int32(0),) * 3)

            def _drain_prev():
                for h in _gathers(slot):
                    h.wait()
                for j in range(3):
                    pltpu.sync_copy(
                        rbufs[j].at[slot],
                        outs[j].at[pl.ds((base + r - 2) * Ks[j], Ks[j])])

            pl.when(r >= 2)(_drain_prev)
            for j in range(3):
                K = Ks[j]
                chunk0 = gbufs[j][pl.ds(0, _L)]
                first = jnp.full((_L,), jnp.sum(
                    jnp.where(lane == 0, chunk0, jnp.int32(0))))
                ib = ibufs[j].at[slot]
                for c in range(K // _L):
                    cur = gbufs[j][pl.ds(c * _L, _L)]
                    keep = (lane + (c * _L)) < ws[j]
                    ib[pl.ds(c * _L, _L)] = jnp.where(keep, cur, first)
            for h in _gathers(slot):
                h.start()

        @pl.loop(0, rows_per, step=2)
        def _rows(r):
            _half(r, 0)
            _half(r + 1, 1)

        for slot in (0, 1):
            for h in _gathers(slot):
                h.wait()
            rprev = rows_per - 2 + slot
            for j in range(3):
                pltpu.sync_copy(
                    rbufs[j].at[slot],
                    outs[j].at[pl.ds((base + rprev) * Ks[j], Ks[j])])

    return body(xf, yf, zf, qxf, qyf, qzf, *tables)


def _mlp_body(K, rows_ref, q_ref, w2, b2, w3, b3, o_ref):
    T = q_ref.shape[0]
    C1 = rows_ref.shape[1]
    g = rows_ref[...].reshape(T, K, C1)
    h1 = _lrelu(g - q_ref[...][:, None, :])
    h2 = _lrelu(h1.reshape(T * K, C1) @ w2[...] + b2[...])
    h3 = _lrelu(h2 @ w3[...] + b3[...])
    C3 = h3.shape[1]
    o_ref[...] = jnp.max(h3.reshape(T, K, C3), axis=1)


def _branch_mlp(rows, q, W2, b2, W3, b3, K):
    R = q.shape[0]
    C1 = rows.shape[1]
    C2 = W2.shape[1]
    C3 = W3.shape[1]
    T = 2048 // K
    return pl.pallas_call(
        functools.partial(_mlp_body, K),
        grid=(R // T,),
        in_specs=[pl.BlockSpec((T * K, C1), lambda i: (i, 0)),
                  pl.BlockSpec((T, C1), lambda i: (i, 0)),
                  pl.BlockSpec((C1, C2), lambda i: (0, 0)),
                  pl.BlockSpec((1, C2), lambda i: (0, 0)),
                  pl.BlockSpec((C2, C3), lambda i: (0, 0)),
                  pl.BlockSpec((1, C3), lambda i: (0, 0))],
        out_specs=pl.BlockSpec((T, C3), lambda i: (i, 0)),
        out_shape=jax.ShapeDtypeStruct((R, C3), jnp.float32),
    )(rows, q, W2, b2.reshape(1, C2), W3, b3.reshape(1, C3))


def _tail_body(xyz_ref, pts_ref, w31, b31, w32, b32, w33, b33,
               wd1, bd1, wd2, bd2, wd3, bd3, out_ref):
    B, S, _ = xyz_ref.shape
    feat = jnp.concatenate([xyz_ref[...], pts_ref[...]], axis=-1)
    feat = feat.reshape(B * S, feat.shape[-1])
    h = _lrelu(feat @ w31[...] + b31[...])
    h = _lrelu(h @ w32[...] + b32[...])
    h = _lrelu(h @ w33[...] + b33[...])
    net = jnp.max(h.reshape(B, S, h.shape[-1]), axis=1)
    net = _lrelu(net @ wd1[...] + bd1[...])
    net = _lrelu(net @ wd2[...] + bd2[...])
    logits = net @ wd3[...] + bd3[...]
    m = jnp.max(logits, axis=-1, keepdims=True)
    e = jnp.exp(logits - m)
    out_ref[...] = e / jnp.sum(e, axis=-1, keepdims=True)


def _tail(xyz2, pts2, params):
    (w31, b31), (w32, b32), (w33, b33) = params['l3']
    (wd1, bd1), = params['d1']
    (wd2, bd2), = params['d2']
    wd3, bd3 = params['d3']
    weights = (w31, b31[None, :], w32, b32[None, :], w33, b33[None, :],
               wd1, bd1[None, :], wd2, bd2[None, :], wd3, bd3[None, :])
    return pl.pallas_call(
        _tail_body,
        out_shape=jax.ShapeDtypeStruct((_B, _NCLS), jnp.float32),
    )(xyz2, pts2, *weights)


def _pad4(w):
    return jnp.concatenate([w, jnp.zeros((1, w.shape[1]), w.dtype)], axis=0)


def _sa_layer(xyz, feats, npoint, radii, Ks, branch_params):
    B, N, _ = xyz.shape
    new_xyz = _fps(xyz, npoint)
    S = npoint
    xflat = xyz.reshape(B * N, 3)
    if feats is None:
        pin = jnp.concatenate([xflat, jnp.zeros((B * N, 1), jnp.float32)], -1)
    else:
        pin = jnp.concatenate([xflat, feats], axis=-1)
    qflat = new_xyz.reshape(B * S, 3)
    qpad = jnp.concatenate([qflat, jnp.zeros((B * S, 1), jnp.float32)], -1)

    tables = []
    Qs = []
    for (W1, b1), _, _ in branch_params:
        W1in = _pad4(W1) if feats is None else W1
        tables.append(_mm(pin, W1in, b1))
        Qs.append(_mm(qpad, _pad4(W1[:3]), jnp.zeros((W1.shape[1],), jnp.float32)))

    xt = jnp.transpose(xyz, (2, 0, 1)).reshape(3, B * N)
    qt = jnp.transpose(new_xyz, (2, 0, 1)).reshape(3, B * S)
    r2s = tuple(float(r) * float(r) for r in radii)
    rows = _sc_select_gather(N, S, tuple(Ks), r2s,
                             xt[0], xt[1], xt[2], qt[0], qt[1], qt[2], tables)
    outs = []
    for j, ((W1, b1), (W2, b2), (W3, b3)) in enumerate(branch_params):
        outs.append(_branch_mlp(rows[j], Qs[j], W2, b2, W3, b3, Ks[j]))
    return new_xyz, jnp.concatenate(outs, axis=-1)


def kernel(input, params):
    xyz1, pts1 = _sa_layer(input, None, 1024, [0.1, 0.2, 0.4], [16, 32, 128],
                           params['l1'])
    xyz2, pts2 = _sa_layer(xyz1, pts1, 512, [0.2, 0.4, 0.8], [32, 64, 128],
                           params['l2'])
    S2 = xyz2.shape[1]
    return _tail(xyz2, pts2.reshape(_B, S2, pts2.shape[-1]), params)


# speedup vs baseline: 1.2495x; 1.0050x over previous
import dataclasses
import functools

import jax
import jax.numpy as jnp
from jax import lax
from jax.experimental import pallas as pl
from jax.experimental.pallas import tpu as pltpu
from jax.experimental.pallas import tpu_sc as plsc

_B = 8
_NCLS = 40
_ALPHA = 0.2
_NW = 32
_L = 16


def _lrelu(x):
    return jnp.where(x >= 0, x, _ALPHA * x)


def _fps_body(npoint, x_ref, y_ref, z_ref, nx_ref, ny_ref, nz_ref):
    X, Y, Z = x_ref[...], y_ref[...], z_ref[...]
    B, N = X.shape
    lane = lax.broadcasted_iota(jnp.int32, (B, N), 1)
    neg = jnp.float32(-3.0e38)
    big = jnp.int32(N)

    def step(t, carry):
        dist, far = carry
        eq = lane == far
        cx = jnp.max(jnp.where(eq, X, neg), axis=1, keepdims=True)
        cy = jnp.max(jnp.where(eq, Y, neg), axis=1, keepdims=True)
        cz = jnp.max(jnp.where(eq, Z, neg), axis=1, keepdims=True)
        nx_ref[pl.ds(t, 1), :] = cx.reshape(1, B)
        ny_ref[pl.ds(t, 1), :] = cy.reshape(1, B)
        nz_ref[pl.ds(t, 1), :] = cz.reshape(1, B)
        dx = X - cx
        dy = Y - cy
        dz = Z - cz
        d = (dx * dx + dy * dy) + dz * dz
        dist = jnp.minimum(dist, d)
        m = jnp.max(dist, axis=1, keepdims=True)
        new_far = jnp.min(jnp.where(dist == m, lane, big), axis=1, keepdims=True)
        return dist, new_far

    init = (jnp.full((B, N), 1e10, dtype=jnp.float32),
            jnp.zeros((B, 1), dtype=jnp.int32))
    lax.fori_loop(0, npoint, step, init, unroll=2)


def _fps(xyz, npoint):
    B, N, _ = xyz.shape
    xt = jnp.transpose(xyz, (2, 0, 1))
    outs = pl.pallas_call(
        functools.partial(_fps_body, npoint),
        out_shape=[jax.ShapeDtypeStruct((npoint, B), jnp.float32)] * 3,
    )(xt[0], xt[1], xt[2])
    return jnp.stack([o.T for o in outs], axis=-1)


def _mm_body(x_ref, w_ref, b_ref, o_ref):
    o_ref[...] = x_ref[...] @ w_ref[...] + b_ref[...]


def _mm(x, w, b, tile=4096):
    R, Cin = x.shape
    C = w.shape[1]
    tile = min(tile, R)
    return pl.pallas_call(
        _mm_body,
        grid=(R // tile,),
        in_specs=[pl.BlockSpec((tile, Cin), lambda i: (i, 0)),
                  pl.BlockSpec((Cin, C), lambda i: (0, 0)),
                  pl.BlockSpec((1, C), lambda i: (0, 0))],
        out_specs=pl.BlockSpec((tile, C), lambda i: (i, 0)),
        out_shape=jax.ShapeDtypeStruct((R, C), jnp.float32),
    )(x, w, b.reshape(1, C))


def _sc_select_gather(N, S, Ks, r2s, xf, yf, zf, qxf, qyf, qzf, tables):
    R = _B * S
    rows_per = R // _NW
    nch = N // _L
    Cs = tuple(int(t.shape[1]) for t in tables)
    mesh = plsc.VectorSubcoreMesh(core_axis_name="c", subcore_axis_name="s")
    out_type = [jax.ShapeDtypeStruct((R * K, C), jnp.float32)
                for K, C in zip(Ks, Cs)]
    scratch = ([pltpu.VMEM((N,), jnp.float32)] * 3
               + [pltpu.VMEM((rows_per,), jnp.float32)] * 3
               + [pltpu.VMEM((K + _L,), jnp.int32) for K in Ks]
               + [pltpu.VMEM((2, K), jnp.int32) for K in Ks]
               + [pltpu.VMEM((2, K, C), jnp.float32) for K, C in zip(Ks, Cs)]
               + [pltpu.SemaphoreType.DMA] * 2)

    cp = pltpu.CompilerParams(needs_layout_passes=False,
                              use_tc_tiling_on_sc=False)

    @functools.partial(pl.kernel, mesh=mesh, out_type=out_type,
                       scratch_types=scratch, compiler_params=cp)
    def body(x_hbm, y_hbm, z_hbm, qx_hbm, qy_hbm, qz_hbm,
             t0_hbm, t1_hbm, t2_hbm, o0_hbm, o1_hbm, o2_hbm,
             Xv, Yv, Zv, Qx, Qy, Qz, g0, g1, g2, i0, i1, i2,
             r0, r1, r2, s0, s1):
        wid = lax.axis_index("s") * 2 + lax.axis_index("c")
        base = wid * rows_per
        b = base // S
        pltpu.sync_copy(x_hbm.at[pl.ds(b * N, N)], Xv)
        pltpu.sync_copy(y_hbm.at[pl.ds(b * N, N)], Yv)
        pltpu.sync_copy(z_hbm.at[pl.ds(b * N, N)], Zv)
        pltpu.sync_copy(qx_hbm.at[pl.ds(base, rows_per)], Qx)
        pltpu.sync_copy(qy_hbm.at[pl.ds(base, rows_per)], Qy)
        pltpu.sync_copy(qz_hbm.at[pl.ds(base, rows_per)], Qz)
        lane = lax.iota(jnp.int32, _L)
        gbufs = (g0, g1, g2)
        ibufs = (i0, i1, i2)
        rbufs = (r0, r1, r2)
        sems = (s0, s1)
        tabs = (t0_hbm, t1_hbm, t2_hbm)
        outs = (o0_hbm, o1_hbm, o2_hbm)

        def _gathers(slot):
            return [pltpu.make_async_copy(
                tabs[j].at[ibufs[j].at[slot]], rbufs[j].at[slot], sems[slot])
                for j in range(3)]

        def _half(r, slot):
            cb = (r // _L) * _L
            sel = lane == (r - cb)
            qx = jnp.full((_L,), jnp.sum(jnp.where(sel, Qx[pl.ds(cb, _L)], 0.0)))
            qy = jnp.full((_L,), jnp.sum(jnp.where(sel, Qy[pl.ds(cb, _L)], 0.0)))
            qz = jnp.full((_L,), jnp.sum(jnp.where(sel, Qz[pl.ds(cb, _L)], 0.0)))

            def chunk(i, ws):
                dx = Xv[pl.ds(i * _L, _L)] - qx
                dy = Yv[pl.ds(i * _L, _L)] - qy
                dz = Zv[pl.ds(i * _L, _L)] - qz
                d = (dx * dx + dy * dy) + dz * dz
                gvec = lane + (i * _L + b * N)
                new_ws = []
                for j in range(3):
                    m = d <= r2s[j]
                    mi = m.astype(jnp.int32)
                    incl = plsc.cumsum(mi)
                    cnt = jnp.max(incl)
                    w = ws[j]
                    pos = incl + (w - 1)
                    pl.when(w < Ks[j])(
                        lambda j=j, pos=pos, m=m: plsc.store_scatter(
                            gbufs[j], [pos], gvec, mask=m))
                    new_ws.append(jnp.where(w < Ks[j], w + cnt, w))
                return tuple(new_ws)

            ws = lax.fori_loop(0, nch, chunk, (jnp.int32(0),) * 3, unroll=2)

            def _drain_prev():
                for h in _gathers(slot):
                    h.wait()
                for j in range(3):
                    pltpu.sync_copy(
                        rbufs[j].at[slot],
                        outs[j].at[pl.ds((base + r - 2) * Ks[j], Ks[j])])

            pl.when(r >= 2)(_drain_prev)
            for j in range(3):
                K = Ks[j]
                chunk0 = gbufs[j][pl.ds(0, _L)]
                first = jnp.full((_L,), jnp.sum(
                    jnp.where(lane == 0, chunk0, jnp.int32(0))))
                ib = ibufs[j].at[slot]
                for c in range(K // _L):
                    cur = gbufs[j][pl.ds(c * _L, _L)]
                    keep = (lane + (c * _L)) < ws[j]
                    ib[pl.ds(c * _L, _L)] = jnp.where(keep, cur, first)
            for h in _gathers(slot):
                h.start()

        @pl.loop(0, rows_per, step=2)
        def _rows(r):
            _half(r, 0)
            _half(r + 1, 1)

        for slot in (0, 1):
            for h in _gathers(slot):
                h.wait()
            rprev = rows_per - 2 + slot
            for j in range(3):
                pltpu.sync_copy(
                    rbufs[j].at[slot],
                    outs[j].at[pl.ds((base + rprev) * Ks[j], Ks[j])])

    return body(xf, yf, zf, qxf, qyf, qzf, *tables)


def _mlp_body(K, rows_ref, q_ref, w2, b2, w3, b3, o_ref):
    T = q_ref.shape[0]
    C1 = rows_ref.shape[1]
    g = rows_ref[...].reshape(T, K, C1)
    h1 = _lrelu(g - q_ref[...][:, None, :])
    h2 = _lrelu(h1.reshape(T * K, C1) @ w2[...] + b2[...])
    h3 = _lrelu(h2 @ w3[...] + b3[...])
    C3 = h3.shape[1]
    o_ref[...] = jnp.max(h3.reshape(T, K, C3), axis=1)


def _branch_mlp(rows, q, W2, b2, W3, b3, K):
    R = q.shape[0]
    C1 = rows.shape[1]
    C2 = W2.shape[1]
    C3 = W3.shape[1]
    T = 2048 // K
    return pl.pallas_call(
        functools.partial(_mlp_body, K),
        grid=(R // T,),
        in_specs=[pl.BlockSpec((T * K, C1), lambda i: (i, 0)),
                  pl.BlockSpec((T, C1), lambda i: (i, 0)),
                  pl.BlockSpec((C1, C2), lambda i: (0, 0)),
                  pl.BlockSpec((1, C2), lambda i: (0, 0)),
                  pl.BlockSpec((C2, C3), lambda i: (0, 0)),
                  pl.BlockSpec((1, C3), lambda i: (0, 0))],
        out_specs=pl.BlockSpec((T, C3), lambda i: (i, 0)),
        out_shape=jax.ShapeDtypeStruct((R, C3), jnp.float32),
    )(rows, q, W2, b2.reshape(1, C2), W3, b3.reshape(1, C3))


def _tail_body(xyz_ref, pts_ref, w31, b31, w32, b32, w33, b33,
               wd1, bd1, wd2, bd2, wd3, bd3, out_ref):
    B, S, _ = xyz_ref.shape
    feat = jnp.concatenate([xyz_ref[...], pts_ref[...]], axis=-1)
    feat = feat.reshape(B * S, feat.shape[-1])
    h = _lrelu(feat @ w31[...] + b31[...])
    h = _lrelu(h @ w32[...] + b32[...])
    h = _lrelu(h @ w33[...] + b33[...])
    net = jnp.max(h.reshape(B, S, h.shape[-1]), axis=1)
    net = _lrelu(net @ wd1[...] + bd1[...])
    net = _lrelu(net @ wd2[...] + bd2[...])
    logits = net @ wd3[...] + bd3[...]
    m = jnp.max(logits, axis=-1, keepdims=True)
    e = jnp.exp(logits - m)
    out_ref[...] = e / jnp.sum(e, axis=-1, keepdims=True)


def _tail(xyz2, pts2, params):
    (w31, b31), (w32, b32), (w33, b33) = params['l3']
    (wd1, bd1), = params['d1']
    (wd2, bd2), = params['d2']
    wd3, bd3 = params['d3']
    weights = (w31, b31[None, :], w32, b32[None, :], w33, b33[None, :],
               wd1, bd1[None, :], wd2, bd2[None, :], wd3, bd3[None, :])
    return pl.pallas_call(
        _tail_body,
        out_shape=jax.ShapeDtypeStruct((_B, _NCLS), jnp.float32),
    )(xyz2, pts2, *weights)


def _pad4(w):
    return jnp.concatenate([w, jnp.zeros((1, w.shape[1]), w.dtype)], axis=0)


def _sa_layer(xyz, feats, npoint, radii, Ks, branch_params):
    B, N, _ = xyz.shape
    new_xyz = _fps(xyz, npoint)
    S = npoint
    xflat = xyz.reshape(B * N, 3)
    if feats is None:
        pin = jnp.concatenate([xflat, jnp.zeros((B * N, 1), jnp.float32)], -1)
    else:
        pin = jnp.concatenate([xflat, feats], axis=-1)
    qflat = new_xyz.reshape(B * S, 3)
    qpad = jnp.concatenate([qflat, jnp.zeros((B * S, 1), jnp.float32)], -1)

    tables = []
    Qs = []
    for (W1, b1), _, _ in branch_params:
        W1in = _pad4(W1) if feats is None else W1
        tables.append(_mm(pin, W1in, b1))
        Qs.append(_mm(qpad, _pad4(W1[:3]), jnp.zeros((W1.shape[1],), jnp.float32)))

    xt = jnp.transpose(xyz, (2, 0, 1)).reshape(3, B * N)
    qt = jnp.transpose(new_xyz, (2, 0, 1)).reshape(3, B * S)
    r2s = tuple(float(r) * float(r) for r in radii)
    rows = _sc_select_gather(N, S, tuple(Ks), r2s,
                             xt[0], xt[1], xt[2], qt[0], qt[1], qt[2], tables)
    outs = []
    for j, ((W1, b1), (W2, b2), (W3, b3)) in enumerate(branch_params):
        outs.append(_branch_mlp(rows[j], Qs[j], W2, b2, W3, b3, Ks[j]))
    return new_xyz, jnp.concatenate(outs, axis=-1)


def kernel(input, params):
    xyz1, pts1 = _sa_layer(input, None, 1024, [0.1, 0.2, 0.4], [16, 32, 128],
                           params['l1'])
    xyz2, pts2 = _sa_layer(xyz1, pts1, 512, [0.2, 0.4, 0.8], [32, 64, 128],
                           params['l2'])
    S2 = xyz2.shape[1]
    return _tail(xyz2, pts2.reshape(_B, S2, pts2.shape[-1]), params)


# speedup vs baseline: 1.2899x; 1.0324x over previous
import dataclasses
import functools

import jax
import jax.numpy as jnp
from jax import lax
from jax.experimental import pallas as pl
from jax.experimental.pallas import tpu as pltpu
from jax.experimental.pallas import tpu_sc as plsc

_B = 8
_NCLS = 40
_ALPHA = 0.2
_NW = 32
_L = 16


def _lrelu(x):
    return jnp.where(x >= 0, x, _ALPHA * x)


def _fps_body(npoint, x_ref, y_ref, z_ref, nx_ref, ny_ref, nz_ref):
    X, Y, Z = x_ref[...], y_ref[...], z_ref[...]
    B, N = X.shape
    lane = lax.broadcasted_iota(jnp.int32, (B, N), 1)
    neg = jnp.float32(-3.0e38)
    big = jnp.int32(N)

    def step(t, carry):
        dist, far = carry
        eq = lane == far
        cx = jnp.max(jnp.where(eq, X, neg), axis=1, keepdims=True)
        cy = jnp.max(jnp.where(eq, Y, neg), axis=1, keepdims=True)
        cz = jnp.max(jnp.where(eq, Z, neg), axis=1, keepdims=True)
        nx_ref[pl.ds(t, 1), :] = cx.reshape(1, B)
        ny_ref[pl.ds(t, 1), :] = cy.reshape(1, B)
        nz_ref[pl.ds(t, 1), :] = cz.reshape(1, B)
        dx = X - cx
        dy = Y - cy
        dz = Z - cz
        d = (dx * dx + dy * dy) + dz * dz
        dist = jnp.minimum(dist, d)
        m = jnp.max(dist, axis=1, keepdims=True)
        new_far = jnp.min(jnp.where(dist == m, lane, big), axis=1, keepdims=True)
        return dist, new_far

    init = (jnp.full((B, N), 1e10, dtype=jnp.float32),
            jnp.zeros((B, 1), dtype=jnp.int32))
    lax.fori_loop(0, npoint, step, init, unroll=2)


def _fps(xyz, npoint):
    B, N, _ = xyz.shape
    xt = jnp.transpose(xyz, (2, 0, 1))
    outs = pl.pallas_call(
        functools.partial(_fps_body, npoint),
        out_shape=[jax.ShapeDtypeStruct((npoint, B), jnp.float32)] * 3,
    )(xt[0], xt[1], xt[2])
    return jnp.stack([o.T for o in outs], axis=-1)


def _mm_body(x_ref, w_ref, b_ref, o_ref):
    o_ref[...] = x_ref[...] @ w_ref[...] + b_ref[...]


def _mm(x, w, b, tile=4096):
    R, Cin = x.shape
    C = w.shape[1]
    tile = min(tile, R)
    return pl.pallas_call(
        _mm_body,
        grid=(R // tile,),
        in_specs=[pl.BlockSpec((tile, Cin), lambda i: (i, 0)),
                  pl.BlockSpec((Cin, C), lambda i: (0, 0)),
                  pl.BlockSpec((1, C), lambda i: (0, 0))],
        out_specs=pl.BlockSpec((tile, C), lambda i: (i, 0)),
        out_shape=jax.ShapeDtypeStruct((R, C), jnp.float32),
    )(x, w, b.reshape(1, C))


def _sc_select_gather(N, S, Ks, r2s, xf, yf, zf, qxf, qyf, qzf, tables, nb=_B):
    R = nb * S
    rows_per = R // _NW
    nch = N // _L
    Cs = tuple(int(t.shape[1]) for t in tables)
    mesh = plsc.VectorSubcoreMesh(core_axis_name="c", subcore_axis_name="s")
    out_type = [jax.ShapeDtypeStruct((R * K, C), jnp.float32)
                for K, C in zip(Ks, Cs)]
    scratch = ([pltpu.VMEM((N,), jnp.float32)] * 3
               + [pltpu.VMEM((rows_per,), jnp.float32)] * 3
               + [pltpu.VMEM((K + _L,), jnp.int32) for K in Ks]
               + [pltpu.VMEM((2, K), jnp.int32) for K in Ks]
               + [pltpu.VMEM((2, K, C), jnp.float32) for K, C in zip(Ks, Cs)]
               + [pltpu.SemaphoreType.DMA] * 2)

    cp = pltpu.CompilerParams(needs_layout_passes=False,
                              use_tc_tiling_on_sc=False)

    @functools.partial(pl.kernel, mesh=mesh, out_type=out_type,
                       scratch_types=scratch, compiler_params=cp)
    def body(x_hbm, y_hbm, z_hbm, qx_hbm, qy_hbm, qz_hbm,
             t0_hbm, t1_hbm, t2_hbm, o0_hbm, o1_hbm, o2_hbm,
             Xv, Yv, Zv, Qx, Qy, Qz, g0, g1, g2, i0, i1, i2,
             r0, r1, r2, s0, s1):
        wid = lax.axis_index("s") * 2 + lax.axis_index("c")
        base = wid * rows_per
        b = base // S
        pltpu.sync_copy(x_hbm.at[pl.ds(b * N, N)], Xv)
        pltpu.sync_copy(y_hbm.at[pl.ds(b * N, N)], Yv)
        pltpu.sync_copy(z_hbm.at[pl.ds(b * N, N)], Zv)
        pltpu.sync_copy(qx_hbm.at[pl.ds(base, rows_per)], Qx)
        pltpu.sync_copy(qy_hbm.at[pl.ds(base, rows_per)], Qy)
        pltpu.sync_copy(qz_hbm.at[pl.ds(base, rows_per)], Qz)
        lane = lax.iota(jnp.int32, _L)
        gbufs = (g0, g1, g2)
        ibufs = (i0, i1, i2)
        rbufs = (r0, r1, r2)
        sems = (s0, s1)
        tabs = (t0_hbm, t1_hbm, t2_hbm)
        outs = (o0_hbm, o1_hbm, o2_hbm)

        def _gathers(slot):
            return [pltpu.make_async_copy(
                tabs[j].at[ibufs[j].at[slot]], rbufs[j].at[slot], sems[slot])
                for j in range(3)]

        def _half(r, slot):
            cb = (r // _L) * _L
            sel = lane == (r - cb)
            qx = jnp.full((_L,), jnp.sum(jnp.where(sel, Qx[pl.ds(cb, _L)], 0.0)))
            qy = jnp.full((_L,), jnp.sum(jnp.where(sel, Qy[pl.ds(cb, _L)], 0.0)))
            qz = jnp.full((_L,), jnp.sum(jnp.where(sel, Qz[pl.ds(cb, _L)], 0.0)))

            def chunk(i, ws):
                dx = Xv[pl.ds(i * _L, _L)] - qx
                dy = Yv[pl.ds(i * _L, _L)] - qy
                dz = Zv[pl.ds(i * _L, _L)] - qz
                d = (dx * dx + dy * dy) + dz * dz
                gvec = lane + (i * _L + b * N)
                new_ws = []
                for j in range(3):
                    m = d <= r2s[j]
                    mi = m.astype(jnp.int32)
                    incl = plsc.cumsum(mi)
                    cnt = jnp.max(incl)
                    w = ws[j]
                    pos = incl + (w - 1)
                    pl.when(w < Ks[j])(
                        lambda j=j, pos=pos, m=m: plsc.store_scatter(
                            gbufs[j], [pos], gvec, mask=m))
                    new_ws.append(jnp.where(w < Ks[j], w + cnt, w))
                return tuple(new_ws)

            ws = lax.fori_loop(0, nch, chunk, (jnp.int32(0),) * 3, unroll=2)

            def _drain_prev():
                for h in _gathers(slot):
                    h.wait()
                for j in range(3):
                    pltpu.sync_copy(
                        rbufs[j].at[slot],
                        outs[j].at[pl.ds((base + r - 2) * Ks[j], Ks[j])])

            pl.when(r >= 2)(_drain_prev)
            for j in range(3):
                K = Ks[j]
                chunk0 = gbufs[j][pl.ds(0, _L)]
                first = jnp.full((_L,), jnp.sum(
                    jnp.where(lane == 0, chunk0, jnp.int32(0))))
                ib = ibufs[j].at[slot]
                for c in range(K // _L):
                    cur = gbufs[j][pl.ds(c * _L, _L)]
                    keep = (lane + (c * _L)) < ws[j]
                    ib[pl.ds(c * _L, _L)] = jnp.where(keep, cur, first)
            for h in _gathers(slot):
                h.start()

        @pl.loop(0, rows_per, step=2)
        def _rows(r):
            _half(r, 0)
            _half(r + 1, 1)

        for slot in (0, 1):
            for h in _gathers(slot):
                h.wait()
            rprev = rows_per - 2 + slot
            for j in range(3):
                pltpu.sync_copy(
                    rbufs[j].at[slot],
                    outs[j].at[pl.ds((base + rprev) * Ks[j], Ks[j])])

    return body(xf, yf, zf, qxf, qyf, qzf, *tables)


def _mlp_body(K, rows_ref, q_ref, w2, b2, w3, b3, o_ref):
    T = q_ref.shape[0]
    C1 = rows_ref.shape[1]
    g = rows_ref[...].reshape(T, K, C1)
    h1 = _lrelu(g - q_ref[...][:, None, :])
    h2 = _lrelu(h1.reshape(T * K, C1) @ w2[...] + b2[...])
    h3 = _lrelu(h2 @ w3[...] + b3[...])
    C3 = h3.shape[1]
    o_ref[...] = jnp.max(h3.reshape(T, K, C3), axis=1)


def _branch_mlp(rows, q, W2, b2, W3, b3, K):
    R = q.shape[0]
    C1 = rows.shape[1]
    C2 = W2.shape[1]
    C3 = W3.shape[1]
    T = 2048 // K
    return pl.pallas_call(
        functools.partial(_mlp_body, K),
        grid=(R // T,),
        in_specs=[pl.BlockSpec((T * K, C1), lambda i: (i, 0)),
                  pl.BlockSpec((T, C1), lambda i: (i, 0)),
                  pl.BlockSpec((C1, C2), lambda i: (0, 0)),
                  pl.BlockSpec((1, C2), lambda i: (0, 0)),
                  pl.BlockSpec((C2, C3), lambda i: (0, 0)),
                  pl.BlockSpec((1, C3), lambda i: (0, 0))],
        out_specs=pl.BlockSpec((T, C3), lambda i: (i, 0)),
        out_shape=jax.ShapeDtypeStruct((R, C3), jnp.float32),
    )(rows, q, W2, b2.reshape(1, C2), W3, b3.reshape(1, C3))


def _tail_body(xyz_ref, pts_ref, w31, b31, w32, b32, w33, b33,
               wd1, bd1, wd2, bd2, wd3, bd3, out_ref):
    B, S, _ = xyz_ref.shape
    feat = jnp.concatenate([xyz_ref[...], pts_ref[...]], axis=-1)
    feat = feat.reshape(B * S, feat.shape[-1])
    h = _lrelu(feat @ w31[...] + b31[...])
    h = _lrelu(h @ w32[...] + b32[...])
    h = _lrelu(h @ w33[...] + b33[...])
    net = jnp.max(h.reshape(B, S, h.shape[-1]), axis=1)
    net = _lrelu(net @ wd1[...] + bd1[...])
    net = _lrelu(net @ wd2[...] + bd2[...])
    logits = net @ wd3[...] + bd3[...]
    m = jnp.max(logits, axis=-1, keepdims=True)
    e = jnp.exp(logits - m)
    out_ref[...] = e / jnp.sum(e, axis=-1, keepdims=True)


def _tail(xyz2, pts2, params):
    (w31, b31), (w32, b32), (w33, b33) = params['l3']
    (wd1, bd1), = params['d1']
    (wd2, bd2), = params['d2']
    wd3, bd3 = params['d3']
    weights = (w31, b31[None, :], w32, b32[None, :], w33, b33[None, :],
               wd1, bd1[None, :], wd2, bd2[None, :], wd3, bd3[None, :])
    return pl.pallas_call(
        _tail_body,
        out_shape=jax.ShapeDtypeStruct((_B, _NCLS), jnp.float32),
    )(xyz2, pts2, *weights)


def _pad4(w):
    return jnp.concatenate([w, jnp.zeros((1, w.shape[1]), w.dtype)], axis=0)


def _sa_layer(xyz, feats, npoint, radii, Ks, branch_params):
    B, N, _ = xyz.shape
    new_xyz = _fps(xyz, npoint)
    S = npoint
    xflat = xyz.reshape(B * N, 3)
    if feats is None:
        pin = jnp.concatenate([xflat, jnp.zeros((B * N, 1), jnp.float32)], -1)
    else:
        pin = jnp.concatenate([xflat, feats], axis=-1)
    qflat = new_xyz.reshape(B * S, 3)
    qpad = jnp.concatenate([qflat, jnp.zeros((B * S, 1), jnp.float32)], -1)

    tables = []
    Qs = []
    for (W1, b1), _, _ in branch_params:
        W1in = _pad4(W1) if feats is None else W1
        tables.append(_mm(pin, W1in, b1))
        Qs.append(_mm(qpad, _pad4(W1[:3]), jnp.zeros((W1.shape[1],), jnp.float32)))

    xt = jnp.transpose(xyz, (2, 0, 1)).reshape(3, B * N)
    qt = jnp.transpose(new_xyz, (2, 0, 1)).reshape(3, B * S)
    r2s = tuple(float(r) * float(r) for r in radii)
    nh = B // 2
    half_outs = []
    for h in range(2):
        psl = slice(h * nh * N, (h + 1) * nh * N)
        qsl = slice(h * nh * S, (h + 1) * nh * S)
        rows = _sc_select_gather(
            N, S, tuple(Ks), r2s,
            xt[0, psl], xt[1, psl], xt[2, psl],
            qt[0, qsl], qt[1, qsl], qt[2, qsl],
            [t[psl] for t in tables], nb=nh)
        houts = []
        for j, ((W1, b1), (W2, b2), (W3, b3)) in enumerate(branch_params):
            houts.append(_branch_mlp(rows[j], Qs[j][qsl], W2, b2, W3, b3,
                                     Ks[j]))
        half_outs.append(jnp.concatenate(houts, axis=-1))
    return new_xyz, jnp.concatenate(half_outs, axis=0)


def kernel(input, params):
    xyz1, pts1 = _sa_layer(input, None, 1024, [0.1, 0.2, 0.4], [16, 32, 128],
                           params['l1'])
    xyz2, pts2 = _sa_layer(xyz1, pts1, 512, [0.2, 0.4, 0.8], [32, 64, 128],
                           params['l2'])
    S2 = xyz2.shape[1]
    return _tail(xyz2, pts2.reshape(_B, S2, pts2.shape[-1]), params)
